# Initial kernel scaffold; baseline (speedup 1.0000x reference)
import jax
import jax.numpy as jnp
from jax.experimental import pallas as pl


def kernel(all_scores, beam_scores):
    raise NotImplementedError("write your pallas kernel here")


# speedup vs baseline: 51.7887x; 51.7887x over previous
import functools

import jax
import jax.numpy as jnp
from jax.experimental import pallas as pl

_NEG = float("-inf")
_BIG = 2**30


def _topk_body(as_ref, bs_ref, vals_ref, beam_ref, word_ref):
    x = as_ref[...]
    x = x[:, 1:, :]
    b = bs_ref[...]
    x = x + b.reshape(5, 1, 1)

    j_beam = jax.lax.broadcasted_iota(jnp.int32, x.shape, 0)
    j_word = jax.lax.broadcasted_iota(jnp.int32, x.shape, 2)
    jgrid = j_beam * 1000 + j_word

    vals = []
    idxs = []
    for _ in range(10):
        m = jnp.max(jnp.max(x, axis=0), axis=-1)
        cand = jnp.where(x == m[None, :, None], jgrid, _BIG)
        jm = jnp.min(jnp.min(cand, axis=0), axis=-1)
        vals.append(m)
        idxs.append(jm)
        x = jnp.where(jgrid == jm[None, :, None], _NEG, x)

    v = jnp.stack(vals, axis=-1)
    j = jnp.stack(idxs, axis=-1)
    vals_ref[0] = v
    beam_ref[0] = j // 1000
    word_ref[0] = j % 1000


@jax.jit
def kernel(all_scores, beam_scores):
    batch, beam = beam_scores.shape
    t = all_scores.shape[1]
    v = all_scores.shape[2]
    grid = (batch,)
    out_shapes = (
        jax.ShapeDtypeStruct((batch, t - 1, 10), jnp.float32),
        jax.ShapeDtypeStruct((batch, t - 1, 10), jnp.int32),
        jax.ShapeDtypeStruct((batch, t - 1, 10), jnp.int32),
    )
    out_spec = pl.BlockSpec((1, t - 1, 10), lambda b: (b, 0, 0))
    return pl.pallas_call(
        _topk_body,
        grid=grid,
        in_specs=[
            pl.BlockSpec((beam, t, v), lambda b: (b, 0, 0)),
            pl.BlockSpec((1, 1, beam), lambda b: (b, 0, 0)),
        ],
        out_specs=(out_spec, out_spec, out_spec),
        out_shape=out_shapes,
    )(all_scores, beam_scores.reshape(batch, 1, beam))


# speedup vs baseline: 77.3628x; 1.4938x over previous
import functools

import jax
import jax.numpy as jnp
from jax import lax
from jax.experimental import pallas as pl
from jax.experimental.pallas import tpu as pltpu
from jax.experimental.pallas import tpu_sc as plsc

BATCH = 64
BEAM = 5
T = 180
V = 1000
K = 10
LANES = 16
GROUPS = 20
CLAMP = V - LANES
TCHUNK = 8
NCORES = 2
NSUB = 16
NW = NCORES * NSUB
BPW = BATCH // NW

TPAD = 184
NEG = float(jnp.finfo(jnp.float32).min)
BIG = 2**30


def _iota16():
    return lax.iota(jnp.int32, LANES)


def _merge_desc(a, b):
    ak, av = a
    bk = lax.rev(b[0], (0,))
    bv = lax.rev(b[1], (0,))
    take = ak >= bk
    mk = jnp.maximum(ak, bk)
    mv = jnp.where(take, av, bv)
    return plsc.sort_key_val(mk, mv, descending=True)


def _merge_desc_j(a, b):
    ak, av = a
    bk = lax.rev(b[0], (0,))
    bv = lax.rev(b[1], (0,))
    take = (ak > bk) | ((ak == bk) & (av < bv))
    mk = jnp.where(take, ak, bk)
    mv = jnp.where(take, av, bv)
    return plsc.sort_key_val(mk, mv, descending=True)


def _merge_tree(pairs, merge):
    while len(pairs) > 1:
        nxt = []
        for i in range(0, len(pairs) - 1, 2):
            nxt.append(merge(pairs[i], pairs[i + 1]))
        if len(pairs) % 2:
            nxt.append(pairs[-1])
        pairs = nxt
    return pairs[0]


def _sc_body(as_hbm, bs_hbm, out_v, out_b, out_w,
             buf, mkey, bsv, ovals, obeam, oword, sem_in, sem_out):
    wid = lax.axis_index("s") * NCORES + lax.axis_index("c")
    lane = _iota16()
    neg16 = jnp.full((LANES,), NEG, jnp.float32)

    def row_body(tt, carry):
        for g in range(GROUPS):
            kb = g // 4
            q = g % 4
            m = buf[kb, tt, pl.ds(min(q * 256, CLAMP), LANES)]
            for i in range(1, LANES):
                off = min(q * 256 + i * LANES, CLAMP)
                m = jnp.maximum(m, buf[kb, tt, pl.ds(off, LANES)])
            mkey[g] = m + bsv[kb]

        leaves = []
        for g in range(GROUPS):
            ck, cv = plsc.sort_key_val(mkey[g], lane + g * LANES,
                                       descending=True)
            leaves.append((ck, cv))
        _, colids = _merge_tree(leaves, _merge_desc)

        g_ = lax.shift_right_logical(colids, 4)
        ln = colids & 15
        kv = lax.shift_right_logical(g_, 2)
        qv = g_ & 3
        base = qv * 256
        ttv = jnp.zeros((LANES,), jnp.int32) + tt
        bsg = bsv[0]
        for k in range(1, BEAM):
            bsg = jnp.where(kv == k, bsv[k], bsg)
        cpairs = []
        for i in range(LANES):
            off = jnp.minimum(base + i * LANES, CLAMP)
            pos = off + ln
            cval = plsc.load_gather(buf, [kv, ttv, pos]) + bsg
            if i == 14:
                cval = jnp.where((qv == 3) & (ln < 8), NEG, cval)
            elif i == 15:
                cval = jnp.where(qv == 3, NEG, cval)
            cpairs.append(plsc.sort_key_val(cval, kv * V + pos,
                                            descending=True))

        x, jv = _merge_tree(cpairs, _merge_desc_j)
        outv = neg16
        outj = jnp.zeros((LANES,), jnp.int32)
        for i in range(K):
            m = jnp.max(x)
            jm = jnp.min(jnp.where(x == m, jv, BIG))
            outv = jnp.where(lane == i, m, outv)
            outj = jnp.where(lane == i, jm, outj)
            x = jnp.where(jv == jm, NEG, x)
        beam = outj // V
        ovals[tt] = outv
        obeam[tt] = beam
        oword[tt] = outj - beam * V
        return carry

    def chunk_body(c, batch):
        t0 = pl.multiple_of(jnp.minimum(c * TCHUNK, TPAD - TCHUNK), TCHUNK)
        copies = []
        for k in range(BEAM):
            copies.append(pltpu.async_copy(
                as_hbm.at[batch * BEAM + k, pl.ds(t0, TCHUNK), :],
                buf.at[k], sem_in))
        for d in copies:
            d.wait()
        lax.fori_loop(0, TCHUNK, row_body, 0)
        outs = [
            pltpu.async_copy(ovals, out_v.at[batch, pl.ds(t0, TCHUNK), :],
                             sem_out),
            pltpu.async_copy(obeam, out_b.at[batch, pl.ds(t0, TCHUNK), :],
                             sem_out),
            pltpu.async_copy(oword, out_w.at[batch, pl.ds(t0, TCHUNK), :],
                             sem_out),
        ]
        for d in outs:
            d.wait()
        return batch

    def batch_body(bi, _):
        batch = wid * BPW + bi
        pltpu.sync_copy(bs_hbm.at[batch], bsv)
        lax.fori_loop(0, TPAD // TCHUNK, chunk_body, batch)
        return 0

    lax.fori_loop(0, BPW, batch_body, 0)


@jax.jit
def kernel(all_scores, beam_scores):
    bs_pad = jnp.broadcast_to(beam_scores[:, :, None],
                              (BATCH, BEAM, LANES)).astype(jnp.float32)
    mesh = plsc.VectorSubcoreMesh(core_axis_name="c", subcore_axis_name="s",
                                  num_cores=NCORES, num_subcores=NSUB)
    out_type = (
        jax.ShapeDtypeStruct((BATCH, TPAD, LANES), jnp.float32),
        jax.ShapeDtypeStruct((BATCH, TPAD, LANES), jnp.int32),
        jax.ShapeDtypeStruct((BATCH, TPAD, LANES), jnp.int32),
    )
    run = pl.kernel(
        _sc_body,
        out_type,
        mesh=mesh,
        compiler_params=pltpu.CompilerParams(needs_layout_passes=False),
        scratch_types=[
            pltpu.VMEM((BEAM, TCHUNK, V), jnp.float32),
            pltpu.VMEM((GROUPS, LANES), jnp.float32),
            pltpu.VMEM((BEAM, LANES), jnp.float32),
            pltpu.VMEM((TCHUNK, LANES), jnp.float32),
            pltpu.VMEM((TCHUNK, LANES), jnp.int32),
            pltpu.VMEM((TCHUNK, LANES), jnp.int32),
            pltpu.SemaphoreType.DMA,
            pltpu.SemaphoreType.DMA,
        ],
    )
    vals, beams, words = run(all_scores, bs_pad)
    return (vals[:, 1:T, :K], beams[:, 1:T, :K], words[:, 1:T, :K])


# speedup vs baseline: 81.9799x; 1.0597x over previous
import functools

import jax
import jax.numpy as jnp
from jax import lax
from jax.experimental import pallas as pl
from jax.experimental.pallas import tpu as pltpu
from jax.experimental.pallas import tpu_sc as plsc

BATCH = 64
BEAM = 5
T = 180
V = 1000
K = 10
LANES = 16
GROUPS = 20
CLAMP = V - LANES
TCHUNK = 8
NCORES = 2
NSUB = 16
NW = NCORES * NSUB
BPW = BATCH // NW

TPAD = 184
NEG = float(jnp.finfo(jnp.float32).min)
BIG = 2**30


def _iota16():
    return lax.iota(jnp.int32, LANES)


def _merge_desc(a, b):
    ak, av = a
    bk = lax.rev(b[0], (0,))
    bv = lax.rev(b[1], (0,))
    take = ak >= bk
    mk = jnp.maximum(ak, bk)
    mv = jnp.where(take, av, bv)
    return plsc.sort_key_val(mk, mv, descending=True)


def _merge_desc_j(a, b):
    ak, av = a
    bk = lax.rev(b[0], (0,))
    bv = lax.rev(b[1], (0,))
    take = (ak > bk) | ((ak == bk) & (av < bv))
    mk = jnp.where(take, ak, bk)
    mv = jnp.where(take, av, bv)
    return plsc.sort_key_val(mk, mv, descending=True)


def _merge_tree(pairs, merge):
    while len(pairs) > 1:
        nxt = []
        for i in range(0, len(pairs) - 1, 2):
            nxt.append(merge(pairs[i], pairs[i + 1]))
        if len(pairs) % 2:
            nxt.append(pairs[-1])
        pairs = nxt
    return pairs[0]


def _sc_body(as_hbm, bs_hbm, out_v, out_b, out_w,
             buf, bsv, ovals, obeam, oword, sem_in, sem_out):
    wid = lax.axis_index("s") * NCORES + lax.axis_index("c")
    lane = _iota16()
    neg16 = jnp.full((LANES,), NEG, jnp.float32)

    def row_one(tt):
        partials = []
        for kb in range(BEAM):
            leaves = []
            for q in range(4):
                g = kb * 4 + q
                m0 = buf[kb, tt, pl.ds(q * 256, LANES)]
                m1 = buf[kb, tt, pl.ds(q * 256 + LANES, LANES)]
                for i in range(2, LANES, 2):
                    m0 = jnp.maximum(
                        m0, buf[kb, tt, pl.ds(min(q * 256 + i * LANES,
                                                  CLAMP), LANES)])
                    m1 = jnp.maximum(
                        m1, buf[kb, tt, pl.ds(min(q * 256 + (i + 1) * LANES,
                                                  CLAMP), LANES)])
                leaves.append(plsc.sort_key_val(
                    jnp.maximum(m0, m1) + bsv[kb], lane + g * LANES,
                    descending=True))
            partials.append(_merge_tree(leaves, _merge_desc))
        _, colids = _merge_tree(partials, _merge_desc)

        g_ = lax.shift_right_logical(colids, 4)
        ln = colids & 15
        kv = lax.shift_right_logical(g_, 2)
        qv = g_ & 3
        base = qv * 256
        ttv = jnp.zeros((LANES,), jnp.int32) + tt
        bsg = bsv[0]
        for k in range(1, BEAM):
            bsg = jnp.where(kv == k, bsv[k], bsg)
        cpairs = []
        for i in range(LANES):
            off = jnp.minimum(base + i * LANES, CLAMP)
            pos = off + ln
            cval = plsc.load_gather(buf, [kv, ttv, pos]) + bsg
            if i == 14:
                cval = jnp.where((qv == 3) & (ln < 8), NEG, cval)
            elif i == 15:
                cval = jnp.where(qv == 3, NEG, cval)
            cpairs.append(plsc.sort_key_val(cval, kv * V + pos,
                                            descending=True))

        x, jv = _merge_tree(cpairs, _merge_desc_j)
        outv = neg16
        outj = jnp.zeros((LANES,), jnp.int32)
        for i in range(K):
            m = jnp.max(x)
            jm = jnp.min(jnp.where(x == m, jv, BIG))
            outv = jnp.where(lane == i, m, outv)
            outj = jnp.where(lane == i, jm, outj)
            x = jnp.where(jv == jm, NEG, x)
        beam = outj // V
        ovals[tt] = outv
        obeam[tt] = beam
        oword[tt] = outj - beam * V

    def row_pair(u, carry):
        row_one(u)
        return carry

    def chunk_body(c, batch):
        t0 = pl.multiple_of(jnp.minimum(c * TCHUNK, TPAD - TCHUNK), TCHUNK)
        copies = []
        for k in range(BEAM):
            copies.append(pltpu.async_copy(
                as_hbm.at[batch * BEAM + k, pl.ds(t0, TCHUNK), :],
                buf.at[k], sem_in))
        for d in copies:
            d.wait()
        lax.fori_loop(0, TCHUNK, row_pair, 0)
        outs = [
            pltpu.async_copy(ovals, out_v.at[batch, pl.ds(t0, TCHUNK), :],
                             sem_out),
            pltpu.async_copy(obeam, out_b.at[batch, pl.ds(t0, TCHUNK), :],
                             sem_out),
            pltpu.async_copy(oword, out_w.at[batch, pl.ds(t0, TCHUNK), :],
                             sem_out),
        ]
        for d in outs:
            d.wait()
        return batch

    def batch_body(bi, _):
        batch = wid * BPW + bi
        pltpu.sync_copy(bs_hbm.at[batch], bsv)
        lax.fori_loop(0, TPAD // TCHUNK, chunk_body, batch)
        return 0

    lax.fori_loop(0, BPW, batch_body, 0)


@jax.jit
def kernel(all_scores, beam_scores):
    bs_pad = jnp.broadcast_to(beam_scores[:, :, None],
                              (BATCH, BEAM, LANES)).astype(jnp.float32)
    mesh = plsc.VectorSubcoreMesh(core_axis_name="c", subcore_axis_name="s",
                                  num_cores=NCORES, num_subcores=NSUB)
    out_type = (
        jax.ShapeDtypeStruct((BATCH, TPAD, LANES), jnp.float32),
        jax.ShapeDtypeStruct((BATCH, TPAD, LANES), jnp.int32),
        jax.ShapeDtypeStruct((BATCH, TPAD, LANES), jnp.int32),
    )
    run = pl.kernel(
        _sc_body,
        out_type,
        mesh=mesh,
        compiler_params=pltpu.CompilerParams(needs_layout_passes=False),
        scratch_types=[
            pltpu.VMEM((BEAM, TCHUNK, V), jnp.float32),
            pltpu.VMEM((BEAM, LANES), jnp.float32),
            pltpu.VMEM((TCHUNK, LANES), jnp.float32),
            pltpu.VMEM((TCHUNK, LANES), jnp.int32),
            pltpu.VMEM((TCHUNK, LANES), jnp.int32),
            pltpu.SemaphoreType.DMA,
            pltpu.SemaphoreType.DMA,
        ],
    )
    vals, beams, words = run(all_scores, bs_pad)
    return (vals[:, 1:T, :K], beams[:, 1:T, :K], words[:, 1:T, :K])


# speedup vs baseline: 95.5928x; 1.1661x over previous
import functools

import jax
import jax.numpy as jnp
from jax import lax
from jax.experimental import pallas as pl
from jax.experimental.pallas import tpu as pltpu
from jax.experimental.pallas import tpu_sc as plsc

BATCH = 64
BEAM = 5
T = 180
V = 1000
K = 10
LANES = 16
GROUPS = 20
CLAMP = V - LANES
TCHUNK = 8
NCORES = 2
NSUB = 16
NW = NCORES * NSUB
BPW = BATCH // NW

TPAD = 184
NEG = float(jnp.finfo(jnp.float32).min)
BIG = 2**30


def _iota16():
    return lax.iota(jnp.int32, LANES)


def _merge_desc(a, b):
    ak, av = a
    bk = lax.rev(b[0], (0,))
    bv = lax.rev(b[1], (0,))
    take = ak >= bk
    mk = jnp.maximum(ak, bk)
    mv = jnp.where(take, av, bv)
    return plsc.sort_key_val(mk, mv, descending=True)


def _merge_desc_j(a, b):
    ak, av = a
    bk = lax.rev(b[0], (0,))
    bv = lax.rev(b[1], (0,))
    take = (ak > bk) | ((ak == bk) & (av < bv))
    mk = jnp.where(take, ak, bk)
    mv = jnp.where(take, av, bv)
    return plsc.sort_key_val(mk, mv, descending=True)


def _merge_tree(pairs, merge):
    while len(pairs) > 1:
        nxt = []
        for i in range(0, len(pairs) - 1, 2):
            nxt.append(merge(pairs[i], pairs[i + 1]))
        if len(pairs) % 2:
            nxt.append(pairs[-1])
        pairs = nxt
    return pairs[0]


def _sc_body(as_hbm, bs_hbm, out_v, out_b, out_w,
             buf, bsv, ovals, obeam, oword, sem_in, sem_out):
    wid = lax.axis_index("s") * NCORES + lax.axis_index("c")
    lane = _iota16()
    neg16 = jnp.full((LANES,), NEG, jnp.float32)

    def row_one(tt):
        partials = []
        for kb in range(BEAM):
            leaves = []
            for q in range(4):
                g = kb * 4 + q
                m0 = buf[kb, tt, pl.ds(q * 256, LANES)]
                m1 = buf[kb, tt, pl.ds(q * 256 + LANES, LANES)]
                for i in range(2, LANES, 2):
                    m0 = jnp.maximum(
                        m0, buf[kb, tt, pl.ds(min(q * 256 + i * LANES,
                                                  CLAMP), LANES)])
                    m1 = jnp.maximum(
                        m1, buf[kb, tt, pl.ds(min(q * 256 + (i + 1) * LANES,
                                                  CLAMP), LANES)])
                leaves.append(plsc.sort_key_val(
                    jnp.maximum(m0, m1) + bsv[kb], lane + g * LANES,
                    descending=True))
            partials.append(_merge_tree(leaves, _merge_desc))
        _, colids = _merge_tree(partials, _merge_desc)

        g_ = lax.shift_right_logical(colids, 4)
        ln = colids & 15
        kv = lax.shift_right_logical(g_, 2)
        qv = g_ & 3
        base = qv * 256
        ttv = jnp.zeros((LANES,), jnp.int32) + tt
        bsg = bsv[0]
        for k in range(1, BEAM):
            bsg = jnp.where(kv == k, bsv[k], bsg)
        cpairs = []
        for i in range(LANES):
            off = jnp.minimum(base + i * LANES, CLAMP)
            pos = off + ln
            cval = plsc.load_gather(buf, [kv, ttv, pos]) + bsg
            if i == 14:
                cval = jnp.where((qv == 3) & (ln < 8), NEG, cval)
            elif i == 15:
                cval = jnp.where(qv == 3, NEG, cval)
            cpairs.append(plsc.sort_key_val(cval, kv * V + pos,
                                            descending=True))

        x, jv = _merge_tree(cpairs, _merge_desc_j)
        beam = jv // V
        ovals[tt] = x
        obeam[tt] = beam
        oword[tt] = jv - beam * V

    def row_pair(u, carry):
        row_one(u)
        return carry

    def chunk_body(c, batch):
        t0 = pl.multiple_of(jnp.minimum(c * TCHUNK, TPAD - TCHUNK), TCHUNK)
        copies = []
        for k in range(BEAM):
            copies.append(pltpu.async_copy(
                as_hbm.at[batch * BEAM + k, pl.ds(t0, TCHUNK), :],
                buf.at[k], sem_in))
        for d in copies:
            d.wait()
        lax.fori_loop(0, TCHUNK, row_pair, 0)
        outs = [
            pltpu.async_copy(ovals, out_v.at[batch, pl.ds(t0, TCHUNK), :],
                             sem_out),
            pltpu.async_copy(obeam, out_b.at[batch, pl.ds(t0, TCHUNK), :],
                             sem_out),
            pltpu.async_copy(oword, out_w.at[batch, pl.ds(t0, TCHUNK), :],
                             sem_out),
        ]
        for d in outs:
            d.wait()
        return batch

    def batch_body(bi, _):
        batch = wid * BPW + bi
        pltpu.sync_copy(bs_hbm.at[batch], bsv)
        lax.fori_loop(0, TPAD // TCHUNK, chunk_body, batch)
        return 0

    lax.fori_loop(0, BPW, batch_body, 0)


@jax.jit
def kernel(all_scores, beam_scores):
    bs_pad = jnp.broadcast_to(beam_scores[:, :, None],
                              (BATCH, BEAM, LANES)).astype(jnp.float32)
    mesh = plsc.VectorSubcoreMesh(core_axis_name="c", subcore_axis_name="s",
                                  num_cores=NCORES, num_subcores=NSUB)
    out_type = (
        jax.ShapeDtypeStruct((BATCH, TPAD, LANES), jnp.float32),
        jax.ShapeDtypeStruct((BATCH, TPAD, LANES), jnp.int32),
        jax.ShapeDtypeStruct((BATCH, TPAD, LANES), jnp.int32),
    )
    run = pl.kernel(
        _sc_body,
        out_type,
        mesh=mesh,
        compiler_params=pltpu.CompilerParams(needs_layout_passes=False),
        scratch_types=[
            pltpu.VMEM((BEAM, TCHUNK, V), jnp.float32),
            pltpu.VMEM((BEAM, LANES), jnp.float32),
            pltpu.VMEM((TCHUNK, LANES), jnp.float32),
            pltpu.VMEM((TCHUNK, LANES), jnp.int32),
            pltpu.VMEM((TCHUNK, LANES), jnp.int32),
            pltpu.SemaphoreType.DMA,
            pltpu.SemaphoreType.DMA,
        ],
    )
    vals, beams, words = run(all_scores, bs_pad)
    return (vals[:, 1:T, :K], beams[:, 1:T, :K], words[:, 1:T, :K])


# speedup vs baseline: 123.6681x; 1.2937x over previous
import functools

import jax
import jax.numpy as jnp
from jax import lax
from jax.experimental import pallas as pl
from jax.experimental.pallas import tpu as pltpu
from jax.experimental.pallas import tpu_sc as plsc

BATCH = 64
BEAM = 5
T = 180
V = 1000
K = 10
LANES = 16
GROUPS = 20
CLAMP = V - LANES
TCHUNK = 8
NCORES = 2
NSUB = 16
NW = NCORES * NSUB
BPW = BATCH // NW

TPAD = 184
NEG = float(jnp.finfo(jnp.float32).min)
BIG = 2**30


def _iota16():
    return lax.iota(jnp.int32, LANES)


def _merge_desc(a, b):
    ak, av = a
    bk = lax.rev(b[0], (0,))
    bv = lax.rev(b[1], (0,))
    take = ak >= bk
    mk = jnp.maximum(ak, bk)
    mv = jnp.where(take, av, bv)
    return plsc.sort_key_val(mk, mv, descending=True)


def _merge_desc_j(a, b):
    ak, av = a
    bk = lax.rev(b[0], (0,))
    bv = lax.rev(b[1], (0,))
    take = (ak > bk) | ((ak == bk) & (av < bv))
    mk = jnp.where(take, ak, bk)
    mv = jnp.where(take, av, bv)
    return plsc.sort_key_val(mk, mv, descending=True)


def _merge_tree(pairs, merge):
    while len(pairs) > 1:
        nxt = []
        for i in range(0, len(pairs) - 1, 2):
            nxt.append(merge(pairs[i], pairs[i + 1]))
        if len(pairs) % 2:
            nxt.append(pairs[-1])
        pairs = nxt
    return pairs[0]


def _sc_body(as_hbm, bs_hbm, out_v, out_b, out_w,
             buf, bsv0, bsv1, ovals, obeam, oword, sem_in, sem_out):
    wid = lax.axis_index("s") * NCORES + lax.axis_index("c")
    lane = _iota16()
    neg16 = jnp.full((LANES,), NEG, jnp.float32)

    def row_one(tt, pb, bi, par):
        partials = []
        for kb in range(BEAM):
            leaves = []
            for q in range(4):
                g = kb * 4 + q
                m0 = buf[pb + kb, tt, pl.ds(q * 256, LANES)]
                m1 = buf[pb + kb, tt, pl.ds(q * 256 + LANES, LANES)]
                for i in range(2, LANES, 2):
                    m0 = jnp.maximum(
                        m0, buf[pb + kb, tt, pl.ds(min(q * 256 + i * LANES,
                                                       CLAMP), LANES)])
                    m1 = jnp.maximum(
                        m1, buf[pb + kb, tt,
                                pl.ds(min(q * 256 + (i + 1) * LANES,
                                          CLAMP), LANES)])
                bsk = jnp.where(bi == 0, bsall[0][kb], bsall[1][kb])
                leaves.append(plsc.sort_key_val(
                    jnp.maximum(m0, m1) + bsk, lane + g * LANES,
                    descending=True))
            partials.append(_merge_tree(leaves, _merge_desc))
        _, colids = _merge_tree(partials, _merge_desc)

        g_ = lax.shift_right_logical(colids, 4)
        ln = colids & 15
        kv = lax.shift_right_logical(g_, 2)
        qv = g_ & 3
        base = qv * 256
        ttv = jnp.zeros((LANES,), jnp.int32) + tt
        bsg = jnp.where(bi == 0, bsall[0][0], bsall[1][0])
        for k in range(1, BEAM):
            bsg = jnp.where(kv == k,
                            jnp.where(bi == 0, bsall[0][k], bsall[1][k]),
                            bsg)
        cpairs = []
        for i in range(LANES):
            off = jnp.minimum(base + i * LANES, CLAMP)
            pos = off + ln
            cval = plsc.load_gather(buf, [kv + pb, ttv, pos]) + bsg
            if i == 14:
                cval = jnp.where((qv == 3) & (ln < 8), NEG, cval)
            elif i == 15:
                cval = jnp.where(qv == 3, NEG, cval)
            cpairs.append(plsc.sort_key_val(cval, kv * V + pos,
                                            descending=True))

        x, jv = _merge_tree(cpairs, _merge_desc_j)
        beam = jv // V
        ovals[par, tt] = x
        obeam[par, tt] = beam
        oword[par, tt] = jv - beam * V

    NCHUNK = TPAD // TCHUNK
    TOTAL = BPW * NCHUNK

    def in_copies(w):
        bi = w // NCHUNK
        c = w - bi * NCHUNK
        batch = wid * BPW + bi
        t0 = pl.multiple_of(jnp.minimum(c * TCHUNK, TPAD - TCHUNK), TCHUNK)
        pb = (w % 2) * BEAM
        return [(as_hbm.at[batch * BEAM + k, pl.ds(t0, TCHUNK), :],
                 buf.at[pb + k]) for k in range(BEAM)]

    def out_copies(w):
        bi = w // NCHUNK
        c = w - bi * NCHUNK
        batch = wid * BPW + bi
        t0 = pl.multiple_of(jnp.minimum(c * TCHUNK, TPAD - TCHUNK), TCHUNK)
        p = w % 2
        dst = pl.ds(t0, TCHUNK)
        return [(ovals.at[p], out_v.at[batch, dst, :]),
                (obeam.at[p], out_b.at[batch, dst, :]),
                (oword.at[p], out_w.at[batch, dst, :])]

    pltpu.sync_copy(bs_hbm.at[wid * BPW], bsv0)
    pltpu.sync_copy(bs_hbm.at[wid * BPW + 1], bsv1)
    bsall = [[bsv0[k] for k in range(BEAM)],
             [bsv1[k] for k in range(BEAM)]]
    for s, d in in_copies(0):
        pltpu.async_copy(s, d, sem_in)

    def work_body(w, carry):
        nxt = jnp.minimum(w + 1, TOTAL - 1)
        nxt = jnp.where(w + 1 < TOTAL, nxt, w - 1)
        for s, d in in_copies(nxt):
            pltpu.async_copy(s, d, sem_in)
        for s, d in in_copies(w):
            pltpu.make_async_copy(s, d, sem_in).wait()

        @pl.when(w > 0)
        def _():
            for s, d in out_copies(w - 1):
                pltpu.make_async_copy(s, d, sem_out).wait()

        bi = w // NCHUNK
        par = w % 2
        pb = par * BEAM

        def row_body(tt, c2):
            row_one(tt, pb, bi, par)
            return c2

        lax.fori_loop(0, TCHUNK, row_body, 0)
        for s, d in out_copies(w):
            pltpu.async_copy(s, d, sem_out)
        return carry

    lax.fori_loop(0, TOTAL, work_body, 0)
    for s, d in in_copies(TOTAL - 2):
        pltpu.make_async_copy(s, d, sem_in).wait()
    for s, d in out_copies(TOTAL - 1):
        pltpu.make_async_copy(s, d, sem_out).wait()


@jax.jit
def kernel(all_scores, beam_scores):
    bs_pad = jnp.broadcast_to(beam_scores[:, :, None],
                              (BATCH, BEAM, LANES)).astype(jnp.float32)
    mesh = plsc.VectorSubcoreMesh(core_axis_name="c", subcore_axis_name="s",
                                  num_cores=NCORES, num_subcores=NSUB)
    out_type = (
        jax.ShapeDtypeStruct((BATCH, TPAD, LANES), jnp.float32),
        jax.ShapeDtypeStruct((BATCH, TPAD, LANES), jnp.int32),
        jax.ShapeDtypeStruct((BATCH, TPAD, LANES), jnp.int32),
    )
    run = pl.kernel(
        _sc_body,
        out_type,
        mesh=mesh,
        compiler_params=pltpu.CompilerParams(needs_layout_passes=False),
        scratch_types=[
            pltpu.VMEM((2 * BEAM, TCHUNK, V), jnp.float32),
            pltpu.VMEM((BEAM, LANES), jnp.float32),
            pltpu.VMEM((BEAM, LANES), jnp.float32),
            pltpu.VMEM((2, TCHUNK, LANES), jnp.float32),
            pltpu.VMEM((2, TCHUNK, LANES), jnp.int32),
            pltpu.VMEM((2, TCHUNK, LANES), jnp.int32),
            pltpu.SemaphoreType.DMA,
            pltpu.SemaphoreType.DMA,
        ],
    )
    vals, beams, words = run(all_scores, bs_pad)
    return (vals[:, 1:T, :K], beams[:, 1:T, :K], words[:, 1:T, :K])
